# baseline (speedup 1.0000x reference)
import functools

import jax
import jax.numpy as jnp
from jax import lax
from jax.experimental import pallas as pl
from jax.experimental.pallas import tpu as pltpu
from jax.experimental.pallas import tpu_sc as plsc

EMB = 32
B = 4096
L = 50
LPAD = 64
NW = 32
ROWS = 2 * B
ROWS_PER_W = ROWS // NW
STEPS = ROWS_PER_W * LPAD // 128


def _make_pool_kernel():
    mesh = plsc.VectorSubcoreMesh(core_axis_name="c", subcore_axis_name="s")

    @functools.partial(
        pl.kernel,
        mesh=mesh,
        out_type=jax.ShapeDtypeStruct((ROWS, EMB), jnp.float32),
        scratch_types=[
            pltpu.VMEM((STEPS, 128), jnp.int32),
            pltpu.VMEM((128,), jnp.int32),
            pltpu.VMEM((128, 128), jnp.float32),
            pltpu.VMEM((ROWS_PER_W, EMB), jnp.float32),
            pltpu.SemaphoreType.DMA,
        ],
        compiler_params=pltpu.CompilerParams(use_tc_tiling_on_sc=True),
    )
    def pool(table_hbm, idx_hbm, out_hbm, idx_v, ridx_v, rows_v, out_v, sem):
        wid = lax.axis_index("s") * 2 + lax.axis_index("c")
        pltpu.sync_copy(idx_hbm.at[wid], idx_v)

        @pl.loop(0, STEPS)
        def _(st):
            for g in range(8):
                ridx_v[pl.ds(g * 16, 16)] = (
                    lax.shift_right_logical(idx_v[st, pl.ds(g * 16, 16)], 2))
            pltpu.async_copy(table_hbm.at[ridx_v], rows_v, sem).wait()
            for half in range(2):
                accs = [jnp.zeros((16,), jnp.float32) for _ in range(8)]
                for j in range(LPAD):
                    e = half * LPAD + j
                    if j % 16 == 0:
                        qv = (idx_v[st, pl.ds(e, 16)] & 3) * EMB
                    q = qv[j % 16]
                    k = j % 4
                    accs[k] = accs[k] + rows_v[e, pl.ds(q, 16)]
                    accs[4 + k] = accs[4 + k] + rows_v[e, pl.ds(q + 16, 16)]
                s = st * 2 + half
                out_v[s, pl.ds(0, 16)] = (accs[0] + accs[1]) + (accs[2] + accs[3])
                out_v[s, pl.ds(16, 16)] = (accs[4] + accs[5]) + (accs[6] + accs[7])

        pltpu.sync_copy(out_v, out_hbm.at[pl.ds(wid * ROWS_PER_W, ROWS_PER_W), :])

    return pool


_pool = _make_pool_kernel()


def _mlp_body(p_ref, il_ref, w1a_ref, w1b_ref, b1_ref, w2_ref, b2_ref, o_ref):
    x0 = p_ref[0] / il_ref[0]
    x1 = p_ref[1] / il_ref[1]
    h = jnp.dot(x0, w1a_ref[...], preferred_element_type=jnp.float32)
    h = h + jnp.dot(x1, w1b_ref[...], preferred_element_type=jnp.float32)
    h = jnp.maximum(h + b1_ref[...], 0.0)
    o_ref[...] = jnp.dot(h, w2_ref[...], preferred_element_type=jnp.float32) + b2_ref[...]


def kernel(data, length, embed_table, W1, b1, W2, b2):
    idx3 = jnp.pad(data.reshape(ROWS, L), ((0, 0), (0, LPAD - L))).reshape(NW, STEPS, 128)
    ctable = embed_table.reshape(-1, 128)
    pooled = _pool(ctable, idx3).reshape(2, B, EMB)
    lenf = length.astype(jnp.float32).reshape(2, B, 1)
    w2p = jnp.pad(W2, ((0, 0), (0, 128 - W2.shape[1])))
    b2p = jnp.pad(b2, (0, 128 - b2.shape[0]))
    out = pl.pallas_call(
        _mlp_body,
        out_shape=jax.ShapeDtypeStruct((B, 128), jnp.float32),
    )(pooled, lenf, W1[:EMB], W1[EMB:], b1.reshape(1, -1),
      w2p, b2p.reshape(1, -1))
    return out[:, :3]
